# Initial kernel scaffold; baseline (speedup 1.0000x reference)
import jax
import jax.numpy as jnp
from jax.experimental import pallas as pl


def kernel(ref_coord, solute_coords, W1, b1, W2, b2, W3, b3, Wp, bp, Ws, bs):
    raise NotImplementedError("write your pallas kernel here")


# speedup vs baseline: 1.1483x; 1.1483x over previous
import functools

import jax
import jax.numpy as jnp
import numpy as np
from jax import lax
from jax.experimental import pallas as pl

B = 1024
N = 8192
D = 3
K = 12
H = 512
SHELL = 12
EV = 2
FLAT = K * D
OUT = SHELL * D * EV
NF = N * D


def _knn_body(ref_ref, sol_ref, out_ref, *, rows):
    sol = sol_ref[...]
    lane = lax.broadcasted_iota(jnp.int32, (rows, NF), 1)
    comp = lane % 3
    rx = ref_ref[:, 0:1]
    ry = ref_ref[:, 1:2]
    rz = ref_ref[:, 2:3]
    refpat = jnp.where(comp == 0, rx, jnp.where(comp == 1, ry, rz))
    local = sol - refpat
    local1 = jnp.concatenate([local[:, 1:], local[:, :1]], axis=1)
    local2 = jnp.concatenate([local[:, 2:], local[:, :2]], axis=1)
    sq = local * local
    sq1 = local1 * local1
    sq2 = local2 * local2
    t = (sq + sq1) + sq2
    inf = jnp.float32(np.inf)
    t = jnp.where(comp == 0, t, inf)

    cols = []
    for _ in range(K):
        m = jnp.min(t, axis=1, keepdims=True)
        eqm = t == m
        pidx = jnp.min(jnp.where(eqm, lane, NF), axis=1, keepdims=True)
        pick = lane == pidx
        zero = jnp.float32(0.0)
        x = jnp.sum(jnp.where(pick, local, zero), axis=1, keepdims=True)
        y = jnp.sum(jnp.where(pick, local1, zero), axis=1, keepdims=True)
        z = jnp.sum(jnp.where(pick, local2, zero), axis=1, keepdims=True)
        cols += [x, y, z]
        t = jnp.where(pick, inf, t)
    out_ref[...] = jnp.concatenate(cols, axis=1)


def _mlp_body(flat_ref, W1_ref, b1_ref, W2_ref, b2_ref, W3_ref, b3_ref,
              Wp_ref, bp_ref, Ws_ref, bs_ref, params_ref, shifts_ref):
    f = flat_ref[...]
    h = jnp.maximum(jnp.dot(f, W1_ref[...]) + b1_ref[...], 0.0)
    h = jnp.maximum(jnp.dot(h, W2_ref[...]) + b2_ref[...], 0.0)
    h = jnp.maximum(jnp.dot(h, W3_ref[...]) + b3_ref[...], 0.0)
    params_ref[...] = jnp.dot(h, Wp_ref[...]) + bp_ref[...]
    shifts_ref[...] = jnp.dot(f, Ws_ref[...]) + bs_ref[...]


@functools.partial(jax.jit, static_argnames=("interpret",))
def kernel(ref_coord, solute_coords, W1, b1, W2, b2, W3, b3, Wp, bp, Ws, bs,
           interpret=False):
    sol_flat = solute_coords.reshape(B, NF)
    rows = 16
    flat = pl.pallas_call(
        functools.partial(_knn_body, rows=rows),
        grid=(B // rows,),
        in_specs=[
            pl.BlockSpec((rows, D), lambda i: (i, 0)),
            pl.BlockSpec((rows, NF), lambda i: (i, 0)),
        ],
        out_specs=pl.BlockSpec((rows, FLAT), lambda i: (i, 0)),
        out_shape=jax.ShapeDtypeStruct((B, FLAT), jnp.float32),
        interpret=interpret,
    )(ref_coord, sol_flat)

    params, shifts = pl.pallas_call(
        _mlp_body,
        out_shape=(
            jax.ShapeDtypeStruct((B, OUT), jnp.float32),
            jax.ShapeDtypeStruct((B, OUT), jnp.float32),
        ),
        interpret=interpret,
    )(flat, W1, b1.reshape(1, H), W2, b2.reshape(1, H), W3, b3.reshape(1, H),
      Wp, bp.reshape(1, OUT), Ws, bs.reshape(1, OUT))
    return (params.reshape(B, SHELL, D, EV), shifts.reshape(B, SHELL, D, EV))
